# Initial kernel scaffold; baseline (speedup 1.0000x reference)
import jax
import jax.numpy as jnp
from jax.experimental import pallas as pl


def kernel(x, adj, W0, a0, W1, a1, W2, a2, W3, a3, W_out, a_out):
    raise NotImplementedError("write your pallas kernel here")


# speedup vs baseline: 1.3493x; 1.3493x over previous
import jax
import jax.numpy as jnp
from jax.experimental import pallas as pl

N = 4096
IN_F = 256
HID = 128
HEADS = 4
NEG = 0.2
NEG_FILL = -9e15

BLK_A = 512
BLK_B = 128


def _proj_kernel(x_ref, w_ref, asrc_ref, adst_ref, wh_ref, s_ref, d_ref):
    wh = jnp.dot(x_ref[...], w_ref[...], preferred_element_type=jnp.float32)
    wh_ref[...] = wh
    s_ref[...] = jnp.dot(wh, asrc_ref[...], preferred_element_type=jnp.float32)
    d_ref[...] = jnp.dot(wh, adst_ref[...], preferred_element_type=jnp.float32)


def _attn1_kernel(adj_ref, wh_ref, s_ref, dt_ref, wout_ref, aso_ref, ado_ref,
                  who_ref, so_ref, do_ref):
    adj = adj_ref[...]
    h_parts = []
    for hd in range(HEADS):
        e = s_ref[:, hd:hd + 1] + dt_ref[hd:hd + 1, :]
        e = jnp.where(e >= 0, e, NEG * e)
        e = jnp.where(adj > 0, e, jnp.float32(NEG_FILL))
        m = jnp.max(e, axis=1, keepdims=True)
        p = jnp.exp(e - m)
        attn = p / jnp.sum(p, axis=1, keepdims=True)
        out = jnp.dot(attn, wh_ref[:, hd * HID:(hd + 1) * HID],
                      preferred_element_type=jnp.float32)
        out = jnp.where(out > 0, out, (jnp.exp(out) - 1.0))
        h_parts.append(out)
    hblk = jnp.concatenate(h_parts, axis=1)
    who = jnp.dot(hblk, wout_ref[...], preferred_element_type=jnp.float32)
    who_ref[...] = who
    so_ref[...] = jnp.dot(who, aso_ref[...], preferred_element_type=jnp.float32)
    do_ref[...] = jnp.dot(who, ado_ref[...], preferred_element_type=jnp.float32)


def _attn2_kernel(adj_ref, who_ref, so_ref, dot_ref, out_ref):
    adj = adj_ref[...]
    e = so_ref[...] + dot_ref[...]
    e = jnp.where(e >= 0, e, NEG * e)
    e = jnp.where(adj > 0, e, jnp.float32(NEG_FILL))
    m = jnp.max(e, axis=1, keepdims=True)
    p = jnp.exp(e - m)
    attn = p / jnp.sum(p, axis=1, keepdims=True)
    out = jnp.dot(attn, who_ref[...], preferred_element_type=jnp.float32)
    out = jnp.where(out > 0, out, (jnp.exp(out) - 1.0))
    m2 = jnp.max(out, axis=1, keepdims=True)
    z = out - m2
    out_ref[...] = z - jnp.log(jnp.sum(jnp.exp(z), axis=1, keepdims=True))


def kernel(x, adj, W0, a0, W1, a1, W2, a2, W3, a3, W_out, a_out):
    f32 = jnp.float32
    W_cat = jnp.concatenate([W0, W1, W2, W3], axis=1)
    src_cols, dst_cols = [], []
    for i, a in enumerate((a0, a1, a2, a3)):
        top = jnp.zeros((i * HID, 1), f32)
        bot = jnp.zeros(((HEADS - 1 - i) * HID, 1), f32)
        src_cols.append(jnp.concatenate([top, a[:HID], bot], axis=0))
        dst_cols.append(jnp.concatenate([top, a[HID:], bot], axis=0))
    A_src = jnp.concatenate(src_cols, axis=1)
    A_dst = jnp.concatenate(dst_cols, axis=1)

    FH = HEADS * HID
    wh_cat, s_all, d_all = pl.pallas_call(
        _proj_kernel,
        grid=(N // BLK_A,),
        in_specs=[
            pl.BlockSpec((BLK_A, IN_F), lambda i: (i, 0)),
            pl.BlockSpec((IN_F, FH), lambda i: (0, 0)),
            pl.BlockSpec((FH, HEADS), lambda i: (0, 0)),
            pl.BlockSpec((FH, HEADS), lambda i: (0, 0)),
        ],
        out_specs=[
            pl.BlockSpec((BLK_A, FH), lambda i: (i, 0)),
            pl.BlockSpec((BLK_A, HEADS), lambda i: (i, 0)),
            pl.BlockSpec((BLK_A, HEADS), lambda i: (i, 0)),
        ],
        out_shape=[
            jax.ShapeDtypeStruct((N, FH), f32),
            jax.ShapeDtypeStruct((N, HEADS), f32),
            jax.ShapeDtypeStruct((N, HEADS), f32),
        ],
    )(x, W_cat, A_src, A_dst)

    dt_all = d_all.T
    wh_out, s_out, d_out = pl.pallas_call(
        _attn1_kernel,
        grid=(N // BLK_B,),
        in_specs=[
            pl.BlockSpec((BLK_B, N), lambda i: (i, 0)),
            pl.BlockSpec((N, FH), lambda i: (0, 0)),
            pl.BlockSpec((BLK_B, HEADS), lambda i: (i, 0)),
            pl.BlockSpec((HEADS, N), lambda i: (0, 0)),
            pl.BlockSpec((FH, HID), lambda i: (0, 0)),
            pl.BlockSpec((HID, 1), lambda i: (0, 0)),
            pl.BlockSpec((HID, 1), lambda i: (0, 0)),
        ],
        out_specs=[
            pl.BlockSpec((BLK_B, HID), lambda i: (i, 0)),
            pl.BlockSpec((BLK_B, 1), lambda i: (i, 0)),
            pl.BlockSpec((BLK_B, 1), lambda i: (i, 0)),
        ],
        out_shape=[
            jax.ShapeDtypeStruct((N, HID), f32),
            jax.ShapeDtypeStruct((N, 1), f32),
            jax.ShapeDtypeStruct((N, 1), f32),
        ],
    )(adj, wh_cat, s_all, dt_all, W_out, a_out[:HID], a_out[HID:])

    dt_out = d_out.T
    out = pl.pallas_call(
        _attn2_kernel,
        grid=(N // BLK_B,),
        in_specs=[
            pl.BlockSpec((BLK_B, N), lambda i: (i, 0)),
            pl.BlockSpec((N, HID), lambda i: (0, 0)),
            pl.BlockSpec((BLK_B, 1), lambda i: (i, 0)),
            pl.BlockSpec((1, N), lambda i: (0, 0)),
        ],
        out_specs=pl.BlockSpec((BLK_B, HID), lambda i: (i, 0)),
        out_shape=jax.ShapeDtypeStruct((N, HID), f32),
    )(adj, wh_out, s_out, dt_out)
    return out


# speedup vs baseline: 1.6281x; 1.2066x over previous
import jax
import jax.numpy as jnp
from jax.experimental import pallas as pl

N = 4096
IN_F = 256
HID = 128
HEADS = 4
NEG = 0.2

BLK_A = 512
BLK_B = 128


def _proj_kernel(x_ref, w_ref, asrc_ref, adst_ref, wh_ref, s_ref, d_ref,
                 csum_ref):
    wh = jnp.dot(x_ref[...], w_ref[...], preferred_element_type=jnp.float32)
    wh_ref[...] = wh
    s_ref[...] = jnp.dot(wh, asrc_ref[...], preferred_element_type=jnp.float32)
    d_ref[...] = jnp.dot(wh, adst_ref[...], preferred_element_type=jnp.float32)

    @pl.when(pl.program_id(0) == 0)
    def _():
        csum_ref[...] = jnp.zeros_like(csum_ref)
    csum_ref[...] += jnp.sum(wh, axis=0, keepdims=True)


def _attn1_kernel(adj_ref, wh_ref, s_ref, dt_ref, whmean_ref, wout_ref,
                  aso_ref, ado_ref, who_ref, so_ref, do_ref, csum_ref):
    adj = adj_ref[...]
    h_parts = []
    for hd in range(HEADS):
        z = s_ref[:, hd:hd + 1] + dt_ref[hd:hd + 1, :]
        p = jnp.exp(jnp.maximum(z, NEG * z)) * adj
        denom = jnp.sum(p, axis=1, keepdims=True)
        agg = jnp.dot(p, wh_ref[:, hd * HID:(hd + 1) * HID],
                      preferred_element_type=jnp.float32)
        out = jnp.where(denom > 0, agg / denom,
                        whmean_ref[:, hd * HID:(hd + 1) * HID])
        out = jnp.where(out > 0, out, jnp.exp(out) - 1.0)
        h_parts.append(out)
    hblk = jnp.concatenate(h_parts, axis=1)
    who = jnp.dot(hblk, wout_ref[...], preferred_element_type=jnp.float32)
    who_ref[...] = who
    so_ref[...] = jnp.dot(who, aso_ref[...], preferred_element_type=jnp.float32)
    do_ref[...] = jnp.dot(who, ado_ref[...], preferred_element_type=jnp.float32)

    @pl.when(pl.program_id(0) == 0)
    def _():
        csum_ref[...] = jnp.zeros_like(csum_ref)
    csum_ref[...] += jnp.sum(who, axis=0, keepdims=True)


def _attn2_kernel(adj_ref, who_ref, so_ref, dot_ref, whomean_ref, out_ref):
    adj = adj_ref[...]
    z = so_ref[...] + dot_ref[...]
    p = jnp.exp(jnp.maximum(z, NEG * z)) * adj
    denom = jnp.sum(p, axis=1, keepdims=True)
    agg = jnp.dot(p, who_ref[...], preferred_element_type=jnp.float32)
    out = jnp.where(denom > 0, agg / denom, whomean_ref[...])
    out = jnp.where(out > 0, out, jnp.exp(out) - 1.0)
    m2 = jnp.max(out, axis=1, keepdims=True)
    zz = out - m2
    out_ref[...] = zz - jnp.log(jnp.sum(jnp.exp(zz), axis=1, keepdims=True))


def kernel(x, adj, W0, a0, W1, a1, W2, a2, W3, a3, W_out, a_out):
    f32 = jnp.float32
    W_cat = jnp.concatenate([W0, W1, W2, W3], axis=1)
    src_cols, dst_cols = [], []
    for i, a in enumerate((a0, a1, a2, a3)):
        top = jnp.zeros((i * HID, 1), f32)
        bot = jnp.zeros(((HEADS - 1 - i) * HID, 1), f32)
        src_cols.append(jnp.concatenate([top, a[:HID], bot], axis=0))
        dst_cols.append(jnp.concatenate([top, a[HID:], bot], axis=0))
    A_src = jnp.concatenate(src_cols, axis=1)
    A_dst = jnp.concatenate(dst_cols, axis=1)

    FH = HEADS * HID
    wh_cat, s_all, d_all, wh_csum = pl.pallas_call(
        _proj_kernel,
        grid=(N // BLK_A,),
        in_specs=[
            pl.BlockSpec((BLK_A, IN_F), lambda i: (i, 0)),
            pl.BlockSpec((IN_F, FH), lambda i: (0, 0)),
            pl.BlockSpec((FH, HEADS), lambda i: (0, 0)),
            pl.BlockSpec((FH, HEADS), lambda i: (0, 0)),
        ],
        out_specs=[
            pl.BlockSpec((BLK_A, FH), lambda i: (i, 0)),
            pl.BlockSpec((BLK_A, HEADS), lambda i: (i, 0)),
            pl.BlockSpec((BLK_A, HEADS), lambda i: (i, 0)),
            pl.BlockSpec((1, FH), lambda i: (0, 0)),
        ],
        out_shape=[
            jax.ShapeDtypeStruct((N, FH), f32),
            jax.ShapeDtypeStruct((N, HEADS), f32),
            jax.ShapeDtypeStruct((N, HEADS), f32),
            jax.ShapeDtypeStruct((1, FH), f32),
        ],
    )(x, W_cat, A_src, A_dst)

    dt_all = d_all.T
    wh_mean = wh_csum * (1.0 / N)
    wh_out, s_out, d_out, who_csum = pl.pallas_call(
        _attn1_kernel,
        grid=(N // BLK_B,),
        in_specs=[
            pl.BlockSpec((BLK_B, N), lambda i: (i, 0)),
            pl.BlockSpec((N, FH), lambda i: (0, 0)),
            pl.BlockSpec((BLK_B, HEADS), lambda i: (i, 0)),
            pl.BlockSpec((HEADS, N), lambda i: (0, 0)),
            pl.BlockSpec((1, FH), lambda i: (0, 0)),
            pl.BlockSpec((FH, HID), lambda i: (0, 0)),
            pl.BlockSpec((HID, 1), lambda i: (0, 0)),
            pl.BlockSpec((HID, 1), lambda i: (0, 0)),
        ],
        out_specs=[
            pl.BlockSpec((BLK_B, HID), lambda i: (i, 0)),
            pl.BlockSpec((BLK_B, 1), lambda i: (i, 0)),
            pl.BlockSpec((BLK_B, 1), lambda i: (i, 0)),
            pl.BlockSpec((1, HID), lambda i: (0, 0)),
        ],
        out_shape=[
            jax.ShapeDtypeStruct((N, HID), f32),
            jax.ShapeDtypeStruct((N, 1), f32),
            jax.ShapeDtypeStruct((N, 1), f32),
            jax.ShapeDtypeStruct((1, HID), f32),
        ],
    )(adj, wh_cat, s_all, dt_all, wh_mean, W_out, a_out[:HID], a_out[HID:])

    dt_out = d_out.T
    who_mean = who_csum * (1.0 / N)
    out = pl.pallas_call(
        _attn2_kernel,
        grid=(N // BLK_B,),
        in_specs=[
            pl.BlockSpec((BLK_B, N), lambda i: (i, 0)),
            pl.BlockSpec((N, HID), lambda i: (0, 0)),
            pl.BlockSpec((BLK_B, 1), lambda i: (i, 0)),
            pl.BlockSpec((1, N), lambda i: (0, 0)),
            pl.BlockSpec((1, HID), lambda i: (0, 0)),
        ],
        out_specs=pl.BlockSpec((BLK_B, HID), lambda i: (i, 0)),
        out_shape=jax.ShapeDtypeStruct((N, HID), f32),
    )(adj, wh_out, s_out, dt_out, who_mean)
    return out


# speedup vs baseline: 2.0369x; 1.2511x over previous
import jax
import jax.numpy as jnp
from jax.experimental import pallas as pl

N = 4096
IN_F = 256
HID = 128
HEADS = 4
NEG = 0.2

BLK_A = 512
BLK_B = 256


def _proj_kernel(x_ref, w_ref, asrc_ref, adst_ref, wh_ref, s_ref, dt_ref,
                 csum_ref):
    wh = jnp.dot(x_ref[...], w_ref[...], preferred_element_type=jnp.float32)
    wh_ref[...] = wh
    s_ref[...] = jnp.dot(wh, asrc_ref[...], preferred_element_type=jnp.float32)
    d = jnp.dot(wh, adst_ref[...], preferred_element_type=jnp.float32)
    dt_ref[...] = d.T

    @pl.when(pl.program_id(0) == 0)
    def _():
        csum_ref[...] = jnp.zeros_like(csum_ref)
    csum_ref[...] += jnp.sum(wh, axis=0, keepdims=True)


def _attn1_kernel(adj_ref, wh_ref, s_ref, dt_ref, whmean_ref, wout_ref,
                  aso_ref, ado_ref, who_ref, so_ref, dot_ref, csum_ref):
    adj = adj_ref[...]
    h_parts = []
    for hd in range(HEADS):
        z = s_ref[:, hd:hd + 1] + dt_ref[hd:hd + 1, :]
        p = jnp.exp(jnp.maximum(z, NEG * z)) * adj
        denom = jnp.sum(p, axis=1, keepdims=True)
        agg = jnp.dot(p, wh_ref[:, hd * HID:(hd + 1) * HID],
                      preferred_element_type=jnp.float32)
        out = jnp.where(denom > 0, agg / denom,
                        whmean_ref[:, hd * HID:(hd + 1) * HID])
        out = jnp.where(out > 0, out, jnp.exp(out) - 1.0)
        h_parts.append(out)
    hblk = jnp.concatenate(h_parts, axis=1)
    who = jnp.dot(hblk, wout_ref[...], preferred_element_type=jnp.float32)
    who_ref[...] = who
    so_ref[...] = jnp.dot(who, aso_ref[...], preferred_element_type=jnp.float32)
    do = jnp.dot(who, ado_ref[...], preferred_element_type=jnp.float32)
    dot_ref[...] = do.T

    @pl.when(pl.program_id(0) == 0)
    def _():
        csum_ref[...] = jnp.zeros_like(csum_ref)
    csum_ref[...] += jnp.sum(who, axis=0, keepdims=True)


def _attn2_kernel(adj_ref, who_ref, so_ref, dot_ref, whomean_ref, out_ref):
    adj = adj_ref[...]
    z = so_ref[...] + dot_ref[...]
    p = jnp.exp(jnp.maximum(z, NEG * z)) * adj
    denom = jnp.sum(p, axis=1, keepdims=True)
    agg = jnp.dot(p, who_ref[...], preferred_element_type=jnp.float32)
    out = jnp.where(denom > 0, agg / denom, whomean_ref[...])
    out = jnp.where(out > 0, out, jnp.exp(out) - 1.0)
    m2 = jnp.max(out, axis=1, keepdims=True)
    zz = out - m2
    out_ref[...] = zz - jnp.log(jnp.sum(jnp.exp(zz), axis=1, keepdims=True))


def kernel(x, adj, W0, a0, W1, a1, W2, a2, W3, a3, W_out, a_out):
    f32 = jnp.float32
    W_cat = jnp.concatenate([W0, W1, W2, W3], axis=1)
    src_cols, dst_cols = [], []
    for i, a in enumerate((a0, a1, a2, a3)):
        top = jnp.zeros((i * HID, 1), f32)
        bot = jnp.zeros(((HEADS - 1 - i) * HID, 1), f32)
        src_cols.append(jnp.concatenate([top, a[:HID], bot], axis=0))
        dst_cols.append(jnp.concatenate([top, a[HID:], bot], axis=0))
    A_src = jnp.concatenate(src_cols, axis=1)
    A_dst = jnp.concatenate(dst_cols, axis=1)

    FH = HEADS * HID
    wh_cat, s_all, dt_all, wh_csum = pl.pallas_call(
        _proj_kernel,
        grid=(N // BLK_A,),
        in_specs=[
            pl.BlockSpec((BLK_A, IN_F), lambda i: (i, 0)),
            pl.BlockSpec((IN_F, FH), lambda i: (0, 0)),
            pl.BlockSpec((FH, HEADS), lambda i: (0, 0)),
            pl.BlockSpec((FH, HEADS), lambda i: (0, 0)),
        ],
        out_specs=[
            pl.BlockSpec((BLK_A, FH), lambda i: (i, 0)),
            pl.BlockSpec((BLK_A, HEADS), lambda i: (i, 0)),
            pl.BlockSpec((HEADS, BLK_A), lambda i: (0, i)),
            pl.BlockSpec((1, FH), lambda i: (0, 0)),
        ],
        out_shape=[
            jax.ShapeDtypeStruct((N, FH), f32),
            jax.ShapeDtypeStruct((N, HEADS), f32),
            jax.ShapeDtypeStruct((HEADS, N), f32),
            jax.ShapeDtypeStruct((1, FH), f32),
        ],
    )(x, W_cat, A_src, A_dst)

    wh_mean = wh_csum * (1.0 / N)
    wh_out, s_out, dt_out, who_csum = pl.pallas_call(
        _attn1_kernel,
        grid=(N // BLK_B,),
        in_specs=[
            pl.BlockSpec((BLK_B, N), lambda i: (i, 0)),
            pl.BlockSpec((N, FH), lambda i: (0, 0)),
            pl.BlockSpec((BLK_B, HEADS), lambda i: (i, 0)),
            pl.BlockSpec((HEADS, N), lambda i: (0, 0)),
            pl.BlockSpec((1, FH), lambda i: (0, 0)),
            pl.BlockSpec((FH, HID), lambda i: (0, 0)),
            pl.BlockSpec((HID, 1), lambda i: (0, 0)),
            pl.BlockSpec((HID, 1), lambda i: (0, 0)),
        ],
        out_specs=[
            pl.BlockSpec((BLK_B, HID), lambda i: (i, 0)),
            pl.BlockSpec((BLK_B, 1), lambda i: (i, 0)),
            pl.BlockSpec((1, BLK_B), lambda i: (0, i)),
            pl.BlockSpec((1, HID), lambda i: (0, 0)),
        ],
        out_shape=[
            jax.ShapeDtypeStruct((N, HID), f32),
            jax.ShapeDtypeStruct((N, 1), f32),
            jax.ShapeDtypeStruct((1, N), f32),
            jax.ShapeDtypeStruct((1, HID), f32),
        ],
    )(adj, wh_cat, s_all, dt_all, wh_mean, W_out, a_out[:HID], a_out[HID:])

    who_mean = who_csum * (1.0 / N)
    out = pl.pallas_call(
        _attn2_kernel,
        grid=(N // BLK_B,),
        in_specs=[
            pl.BlockSpec((BLK_B, N), lambda i: (i, 0)),
            pl.BlockSpec((N, HID), lambda i: (0, 0)),
            pl.BlockSpec((BLK_B, 1), lambda i: (i, 0)),
            pl.BlockSpec((1, N), lambda i: (0, 0)),
            pl.BlockSpec((1, HID), lambda i: (0, 0)),
        ],
        out_specs=pl.BlockSpec((BLK_B, HID), lambda i: (i, 0)),
        out_shape=jax.ShapeDtypeStruct((N, HID), f32),
    )(adj, wh_out, s_out, dt_out, who_mean)
    return out


# speedup vs baseline: 2.2194x; 1.0896x over previous
import jax
import jax.numpy as jnp
from jax.experimental import pallas as pl

N = 4096
IN_F = 256
HID = 128
HEADS = 4
NEG = 0.2

BLK_A = 512
BLK_B = 256


def _proj_kernel(x_ref, w_ref, asrc_ref, adst_ref,
                 wh_ref, es_ref, fs_ref, edt_ref, fdt_ref, csum_ref):
    wh = jnp.dot(x_ref[...], w_ref[...], preferred_element_type=jnp.float32)
    wh_ref[...] = wh
    s = jnp.dot(wh, asrc_ref[...], preferred_element_type=jnp.float32)
    d = jnp.dot(wh, adst_ref[...], preferred_element_type=jnp.float32)
    es_ref[...] = jnp.exp(s)
    fs_ref[...] = jnp.exp(NEG * s)
    edt_ref[...] = jnp.exp(d).T
    fdt_ref[...] = jnp.exp(NEG * d).T

    @pl.when(pl.program_id(0) == 0)
    def _():
        csum_ref[...] = jnp.zeros_like(csum_ref)
    csum_ref[...] += jnp.sum(wh, axis=0, keepdims=True)


def _attn1_kernel(adj_ref, wh_ref, es_ref, fs_ref, edt_ref, fdt_ref,
                  whmean_ref, wout_ref, aso_ref, ado_ref,
                  who_ref, eso_ref, fso_ref, edot_ref, fdot_ref, csum_ref):
    adj = adj_ref[...]
    h_parts = []
    for hd in range(HEADS):
        a = es_ref[:, hd:hd + 1] * edt_ref[hd:hd + 1, :]
        b = fs_ref[:, hd:hd + 1] * fdt_ref[hd:hd + 1, :]
        p = jnp.maximum(a, b) * adj
        denom = jnp.sum(p, axis=1, keepdims=True)
        agg = jnp.dot(p, wh_ref[:, hd * HID:(hd + 1) * HID],
                      preferred_element_type=jnp.float32)
        out = jnp.where(denom > 0, agg / denom,
                        whmean_ref[:, hd * HID:(hd + 1) * HID])
        out = jnp.where(out > 0, out, jnp.exp(out) - 1.0)
        h_parts.append(out)
    hblk = jnp.concatenate(h_parts, axis=1)
    who = jnp.dot(hblk, wout_ref[...], preferred_element_type=jnp.float32)
    who_ref[...] = who
    so = jnp.dot(who, aso_ref[...], preferred_element_type=jnp.float32)
    do = jnp.dot(who, ado_ref[...], preferred_element_type=jnp.float32)
    eso_ref[...] = jnp.exp(so)
    fso_ref[...] = jnp.exp(NEG * so)
    edot_ref[...] = jnp.exp(do).T
    fdot_ref[...] = jnp.exp(NEG * do).T

    @pl.when(pl.program_id(0) == 0)
    def _():
        csum_ref[...] = jnp.zeros_like(csum_ref)
    csum_ref[...] += jnp.sum(who, axis=0, keepdims=True)


def _attn2_kernel(adj_ref, who_ref, eso_ref, fso_ref, edot_ref, fdot_ref,
                  whomean_ref, out_ref):
    adj = adj_ref[...]
    a = eso_ref[...] * edot_ref[...]
    b = fso_ref[...] * fdot_ref[...]
    p = jnp.maximum(a, b) * adj
    denom = jnp.sum(p, axis=1, keepdims=True)
    agg = jnp.dot(p, who_ref[...], preferred_element_type=jnp.float32)
    out = jnp.where(denom > 0, agg / denom, whomean_ref[...])
    out = jnp.where(out > 0, out, jnp.exp(out) - 1.0)
    m2 = jnp.max(out, axis=1, keepdims=True)
    zz = out - m2
    out_ref[...] = zz - jnp.log(jnp.sum(jnp.exp(zz), axis=1, keepdims=True))


def kernel(x, adj, W0, a0, W1, a1, W2, a2, W3, a3, W_out, a_out):
    f32 = jnp.float32
    W_cat = jnp.concatenate([W0, W1, W2, W3], axis=1)
    src_cols, dst_cols = [], []
    for i, a in enumerate((a0, a1, a2, a3)):
        top = jnp.zeros((i * HID, 1), f32)
        bot = jnp.zeros(((HEADS - 1 - i) * HID, 1), f32)
        src_cols.append(jnp.concatenate([top, a[:HID], bot], axis=0))
        dst_cols.append(jnp.concatenate([top, a[HID:], bot], axis=0))
    A_src = jnp.concatenate(src_cols, axis=1)
    A_dst = jnp.concatenate(dst_cols, axis=1)

    FH = HEADS * HID
    wh_cat, es_all, fs_all, edt_all, fdt_all, wh_csum = pl.pallas_call(
        _proj_kernel,
        grid=(N // BLK_A,),
        in_specs=[
            pl.BlockSpec((BLK_A, IN_F), lambda i: (i, 0)),
            pl.BlockSpec((IN_F, FH), lambda i: (0, 0)),
            pl.BlockSpec((FH, HEADS), lambda i: (0, 0)),
            pl.BlockSpec((FH, HEADS), lambda i: (0, 0)),
        ],
        out_specs=[
            pl.BlockSpec((BLK_A, FH), lambda i: (i, 0)),
            pl.BlockSpec((BLK_A, HEADS), lambda i: (i, 0)),
            pl.BlockSpec((BLK_A, HEADS), lambda i: (i, 0)),
            pl.BlockSpec((HEADS, BLK_A), lambda i: (0, i)),
            pl.BlockSpec((HEADS, BLK_A), lambda i: (0, i)),
            pl.BlockSpec((1, FH), lambda i: (0, 0)),
        ],
        out_shape=[
            jax.ShapeDtypeStruct((N, FH), f32),
            jax.ShapeDtypeStruct((N, HEADS), f32),
            jax.ShapeDtypeStruct((N, HEADS), f32),
            jax.ShapeDtypeStruct((HEADS, N), f32),
            jax.ShapeDtypeStruct((HEADS, N), f32),
            jax.ShapeDtypeStruct((1, FH), f32),
        ],
    )(x, W_cat, A_src, A_dst)

    wh_mean = wh_csum * (1.0 / N)
    wh_out, eso, fso, edot, fdot, who_csum = pl.pallas_call(
        _attn1_kernel,
        grid=(N // BLK_B,),
        in_specs=[
            pl.BlockSpec((BLK_B, N), lambda i: (i, 0)),
            pl.BlockSpec((N, FH), lambda i: (0, 0)),
            pl.BlockSpec((BLK_B, HEADS), lambda i: (i, 0)),
            pl.BlockSpec((BLK_B, HEADS), lambda i: (i, 0)),
            pl.BlockSpec((HEADS, N), lambda i: (0, 0)),
            pl.BlockSpec((HEADS, N), lambda i: (0, 0)),
            pl.BlockSpec((1, FH), lambda i: (0, 0)),
            pl.BlockSpec((FH, HID), lambda i: (0, 0)),
            pl.BlockSpec((HID, 1), lambda i: (0, 0)),
            pl.BlockSpec((HID, 1), lambda i: (0, 0)),
        ],
        out_specs=[
            pl.BlockSpec((BLK_B, HID), lambda i: (i, 0)),
            pl.BlockSpec((BLK_B, 1), lambda i: (i, 0)),
            pl.BlockSpec((BLK_B, 1), lambda i: (i, 0)),
            pl.BlockSpec((1, BLK_B), lambda i: (0, i)),
            pl.BlockSpec((1, BLK_B), lambda i: (0, i)),
            pl.BlockSpec((1, HID), lambda i: (0, 0)),
        ],
        out_shape=[
            jax.ShapeDtypeStruct((N, HID), f32),
            jax.ShapeDtypeStruct((N, 1), f32),
            jax.ShapeDtypeStruct((N, 1), f32),
            jax.ShapeDtypeStruct((1, N), f32),
            jax.ShapeDtypeStruct((1, N), f32),
            jax.ShapeDtypeStruct((1, HID), f32),
        ],
    )(adj, wh_cat, es_all, fs_all, edt_all, fdt_all, wh_mean, W_out,
      a_out[:HID], a_out[HID:])

    who_mean = who_csum * (1.0 / N)
    out = pl.pallas_call(
        _attn2_kernel,
        grid=(N // BLK_B,),
        in_specs=[
            pl.BlockSpec((BLK_B, N), lambda i: (i, 0)),
            pl.BlockSpec((N, HID), lambda i: (0, 0)),
            pl.BlockSpec((BLK_B, 1), lambda i: (i, 0)),
            pl.BlockSpec((BLK_B, 1), lambda i: (i, 0)),
            pl.BlockSpec((1, N), lambda i: (0, 0)),
            pl.BlockSpec((1, N), lambda i: (0, 0)),
            pl.BlockSpec((1, HID), lambda i: (0, 0)),
        ],
        out_specs=pl.BlockSpec((BLK_B, HID), lambda i: (i, 0)),
        out_shape=jax.ShapeDtypeStruct((N, HID), f32),
    )(adj, wh_out, eso, fso, edot, fdot, who_mean)
    return out


# speedup vs baseline: 2.6500x; 1.1940x over previous
import jax
import jax.numpy as jnp
from jax.experimental import pallas as pl

N = 4096
IN_F = 256
HID = 128
HEADS = 4
NEG = 0.2

BLK_A = 512
BLK_B = 256

BF = jnp.bfloat16


def _ext_pack(wh_f32, hid, heads):
    blk = wh_f32.shape[0]
    parts = []
    for h in range(heads):
        parts.append(wh_f32[:, h * hid:(h + 1) * hid].astype(BF))
        parts.append(jnp.ones((blk, 1), BF))
        parts.append(jnp.zeros((blk, hid - 1), BF))
    return jnp.concatenate(parts, axis=1)


def _proj_kernel(x_ref, adj_ref, w_ref, asrc_ref, adst_ref,
                 whext_ref, adjb_ref, es_ref, fs_ref, edt_ref, fdt_ref,
                 csum_ref):
    wh = jnp.dot(x_ref[...], w_ref[...], preferred_element_type=jnp.float32)
    whext_ref[...] = _ext_pack(wh, HID, HEADS)
    adjb_ref[...] = adj_ref[...].astype(BF)
    s = jnp.dot(wh, asrc_ref[...], preferred_element_type=jnp.float32)
    d = jnp.dot(wh, adst_ref[...], preferred_element_type=jnp.float32)
    es_ref[...] = jnp.exp(s).astype(BF)
    fs_ref[...] = jnp.exp(NEG * s).astype(BF)
    edt_ref[...] = jnp.exp(d).astype(BF).T
    fdt_ref[...] = jnp.exp(NEG * d).astype(BF).T

    @pl.when(pl.program_id(0) == 0)
    def _():
        csum_ref[...] = jnp.zeros_like(csum_ref)
    csum_ref[...] += jnp.sum(wh, axis=0, keepdims=True)


def _attn1_kernel(adj_ref, whext_ref, es_ref, fs_ref, edt_ref, fdt_ref,
                  whmean_ref, wout_ref, aso_ref, ado_ref,
                  whoext_ref, eso_ref, fso_ref, edot_ref, fdot_ref, csum_ref):
    adj = adj_ref[...]
    EXTW = 2 * HID
    h_parts = []
    for hd in range(HEADS):
        a = es_ref[:, hd:hd + 1] * edt_ref[hd:hd + 1, :]
        b = fs_ref[:, hd:hd + 1] * fdt_ref[hd:hd + 1, :]
        p = jnp.maximum(a, b) * adj
        agg_ext = jnp.dot(p, whext_ref[:, hd * EXTW:(hd + 1) * EXTW],
                          preferred_element_type=jnp.float32)
        agg = agg_ext[:, :HID]
        denom = agg_ext[:, HID:HID + 1]
        out = jnp.where(denom > 0, agg / denom,
                        whmean_ref[:, hd * HID:(hd + 1) * HID])
        out = jnp.where(out > 0, out, jnp.exp(out) - 1.0)
        h_parts.append(out)
    hblk = jnp.concatenate(h_parts, axis=1)
    who = jnp.dot(hblk, wout_ref[...], preferred_element_type=jnp.float32)
    whoext_ref[...] = _ext_pack(who, HID, 1)
    so = jnp.dot(who, aso_ref[...], preferred_element_type=jnp.float32)
    do = jnp.dot(who, ado_ref[...], preferred_element_type=jnp.float32)
    eso_ref[...] = jnp.exp(so).astype(BF)
    fso_ref[...] = jnp.exp(NEG * so).astype(BF)
    edot_ref[...] = jnp.exp(do).astype(BF).T
    fdot_ref[...] = jnp.exp(NEG * do).astype(BF).T

    @pl.when(pl.program_id(0) == 0)
    def _():
        csum_ref[...] = jnp.zeros_like(csum_ref)
    csum_ref[...] += jnp.sum(who, axis=0, keepdims=True)


def _attn2_kernel(adj_ref, whoext_ref, eso_ref, fso_ref, edot_ref, fdot_ref,
                  whomean_ref, out_ref):
    adj = adj_ref[...]
    a = eso_ref[...] * edot_ref[...]
    b = fso_ref[...] * fdot_ref[...]
    p = jnp.maximum(a, b) * adj
    agg_ext = jnp.dot(p, whoext_ref[...], preferred_element_type=jnp.float32)
    agg = agg_ext[:, :HID]
    denom = agg_ext[:, HID:HID + 1]
    out = jnp.where(denom > 0, agg / denom, whomean_ref[...])
    out = jnp.where(out > 0, out, jnp.exp(out) - 1.0)
    m2 = jnp.max(out, axis=1, keepdims=True)
    zz = out - m2
    out_ref[...] = zz - jnp.log(jnp.sum(jnp.exp(zz), axis=1, keepdims=True))


def kernel(x, adj, W0, a0, W1, a1, W2, a2, W3, a3, W_out, a_out):
    f32 = jnp.float32
    W_cat = jnp.concatenate([W0, W1, W2, W3], axis=1)
    src_cols, dst_cols = [], []
    for i, a in enumerate((a0, a1, a2, a3)):
        top = jnp.zeros((i * HID, 1), f32)
        bot = jnp.zeros(((HEADS - 1 - i) * HID, 1), f32)
        src_cols.append(jnp.concatenate([top, a[:HID], bot], axis=0))
        dst_cols.append(jnp.concatenate([top, a[HID:], bot], axis=0))
    A_src = jnp.concatenate(src_cols, axis=1)
    A_dst = jnp.concatenate(dst_cols, axis=1)

    FH = HEADS * HID
    EXTW = 2 * HID
    whext, adj_bf, es_all, fs_all, edt_all, fdt_all, wh_csum = pl.pallas_call(
        _proj_kernel,
        grid=(N // BLK_A,),
        in_specs=[
            pl.BlockSpec((BLK_A, IN_F), lambda i: (i, 0)),
            pl.BlockSpec((BLK_A, N), lambda i: (i, 0)),
            pl.BlockSpec((IN_F, FH), lambda i: (0, 0)),
            pl.BlockSpec((FH, HEADS), lambda i: (0, 0)),
            pl.BlockSpec((FH, HEADS), lambda i: (0, 0)),
        ],
        out_specs=[
            pl.BlockSpec((BLK_A, HEADS * EXTW), lambda i: (i, 0)),
            pl.BlockSpec((BLK_A, N), lambda i: (i, 0)),
            pl.BlockSpec((BLK_A, HEADS), lambda i: (i, 0)),
            pl.BlockSpec((BLK_A, HEADS), lambda i: (i, 0)),
            pl.BlockSpec((HEADS, BLK_A), lambda i: (0, i)),
            pl.BlockSpec((HEADS, BLK_A), lambda i: (0, i)),
            pl.BlockSpec((1, FH), lambda i: (0, 0)),
        ],
        out_shape=[
            jax.ShapeDtypeStruct((N, HEADS * EXTW), BF),
            jax.ShapeDtypeStruct((N, N), BF),
            jax.ShapeDtypeStruct((N, HEADS), BF),
            jax.ShapeDtypeStruct((N, HEADS), BF),
            jax.ShapeDtypeStruct((HEADS, N), BF),
            jax.ShapeDtypeStruct((HEADS, N), BF),
            jax.ShapeDtypeStruct((1, FH), f32),
        ],
    )(x, adj, W_cat, A_src, A_dst)

    wh_mean = wh_csum * (1.0 / N)
    whoext, eso, fso, edot, fdot, who_csum = pl.pallas_call(
        _attn1_kernel,
        grid=(N // BLK_B,),
        in_specs=[
            pl.BlockSpec((BLK_B, N), lambda i: (i, 0)),
            pl.BlockSpec((N, HEADS * EXTW), lambda i: (0, 0)),
            pl.BlockSpec((BLK_B, HEADS), lambda i: (i, 0)),
            pl.BlockSpec((BLK_B, HEADS), lambda i: (i, 0)),
            pl.BlockSpec((HEADS, N), lambda i: (0, 0)),
            pl.BlockSpec((HEADS, N), lambda i: (0, 0)),
            pl.BlockSpec((1, FH), lambda i: (0, 0)),
            pl.BlockSpec((FH, HID), lambda i: (0, 0)),
            pl.BlockSpec((HID, 1), lambda i: (0, 0)),
            pl.BlockSpec((HID, 1), lambda i: (0, 0)),
        ],
        out_specs=[
            pl.BlockSpec((BLK_B, EXTW), lambda i: (i, 0)),
            pl.BlockSpec((BLK_B, 1), lambda i: (i, 0)),
            pl.BlockSpec((BLK_B, 1), lambda i: (i, 0)),
            pl.BlockSpec((1, BLK_B), lambda i: (0, i)),
            pl.BlockSpec((1, BLK_B), lambda i: (0, i)),
            pl.BlockSpec((1, HID), lambda i: (0, 0)),
        ],
        out_shape=[
            jax.ShapeDtypeStruct((N, EXTW), BF),
            jax.ShapeDtypeStruct((N, 1), BF),
            jax.ShapeDtypeStruct((N, 1), BF),
            jax.ShapeDtypeStruct((1, N), BF),
            jax.ShapeDtypeStruct((1, N), BF),
            jax.ShapeDtypeStruct((1, HID), f32),
        ],
    )(adj_bf, whext, es_all, fs_all, edt_all, fdt_all, wh_mean, W_out,
      a_out[:HID], a_out[HID:])

    who_mean = who_csum * (1.0 / N)
    out = pl.pallas_call(
        _attn2_kernel,
        grid=(N // BLK_B,),
        in_specs=[
            pl.BlockSpec((BLK_B, N), lambda i: (i, 0)),
            pl.BlockSpec((N, EXTW), lambda i: (0, 0)),
            pl.BlockSpec((BLK_B, 1), lambda i: (i, 0)),
            pl.BlockSpec((BLK_B, 1), lambda i: (i, 0)),
            pl.BlockSpec((1, N), lambda i: (0, 0)),
            pl.BlockSpec((1, N), lambda i: (0, 0)),
            pl.BlockSpec((1, HID), lambda i: (0, 0)),
        ],
        out_specs=pl.BlockSpec((BLK_B, HID), lambda i: (i, 0)),
        out_shape=jax.ShapeDtypeStruct((N, HID), jnp.float32),
    )(adj_bf, whoext, eso, fso, edot, fdot, who_mean)
    return out


# speedup vs baseline: 3.3114x; 1.2496x over previous
import jax
import jax.numpy as jnp
from jax.experimental import pallas as pl

N = 4096
IN_F = 256
HID = 128
HEADS = 4
NEG = 0.2

BLK_A = 512
BLK_B = 256

BF = jnp.bfloat16


def _proj_kernel(x_ref, w_ref, a_ref,
                 whext_ref, es_ref, fs_ref, edt_ref, fdt_ref, csum_ref):
    x = x_ref[...]
    ext_parts, s_parts, d_parts, wh_parts = [], [], [], []
    blk = x.shape[0]
    for h in range(HEADS):
        wh = jnp.dot(x, w_ref[:, h * HID:(h + 1) * HID],
                     preferred_element_type=jnp.float32)
        wh_parts.append(wh)
        s_parts.append(jnp.dot(wh, a_ref[:HID, h:h + 1],
                               preferred_element_type=jnp.float32))
        d_parts.append(jnp.dot(wh, a_ref[HID:, h:h + 1],
                               preferred_element_type=jnp.float32))
        ext_parts.append(wh.astype(BF))
        ext_parts.append(jnp.ones((blk, 1), BF))
        ext_parts.append(jnp.zeros((blk, HID - 1), BF))
    whext_ref[...] = jnp.concatenate(ext_parts, axis=1)
    s = jnp.concatenate(s_parts, axis=1)
    d = jnp.concatenate(d_parts, axis=1)
    es_ref[...] = jnp.exp(s).astype(BF)
    fs_ref[...] = jnp.exp(NEG * s).astype(BF)
    edt_ref[...] = jnp.exp(d).astype(BF).T
    fdt_ref[...] = jnp.exp(NEG * d).astype(BF).T

    @pl.when(pl.program_id(0) == 0)
    def _():
        csum_ref[...] = jnp.zeros_like(csum_ref)
    csum_ref[...] += jnp.sum(jnp.concatenate(wh_parts, axis=1), axis=0,
                             keepdims=True)


def _attn1_kernel(adj_ref, whext_ref, es_ref, fs_ref, edt_ref, fdt_ref,
                  csum_in_ref, wout_ref, aout_ref,
                  adjb_ref, whoext_ref, eso_ref, fso_ref, edot_ref, fdot_ref,
                  csum_ref):
    adj = adj_ref[...].astype(BF)
    adjb_ref[...] = adj
    whmean = csum_in_ref[...] * (1.0 / N)
    EXTW = 2 * HID
    h_parts = []
    for hd in range(HEADS):
        a = es_ref[:, hd:hd + 1] * edt_ref[hd:hd + 1, :]
        b = fs_ref[:, hd:hd + 1] * fdt_ref[hd:hd + 1, :]
        p = jnp.maximum(a, b) * adj
        agg_ext = jnp.dot(p, whext_ref[:, hd * EXTW:(hd + 1) * EXTW],
                          preferred_element_type=jnp.float32)
        agg = agg_ext[:, :HID]
        denom = agg_ext[:, HID:HID + 1]
        out = jnp.where(denom > 0, agg / denom,
                        whmean[:, hd * HID:(hd + 1) * HID])
        out = jnp.where(out > 0, out, jnp.exp(out) - 1.0)
        h_parts.append(out)
    hblk = jnp.concatenate(h_parts, axis=1)
    who = jnp.dot(hblk, wout_ref[...], preferred_element_type=jnp.float32)
    blk = who.shape[0]
    whoext_ref[...] = jnp.concatenate(
        [who.astype(BF), jnp.ones((blk, 1), BF), jnp.zeros((blk, HID - 1), BF)],
        axis=1)
    so = jnp.dot(who, aout_ref[:HID, :], preferred_element_type=jnp.float32)
    do = jnp.dot(who, aout_ref[HID:, :], preferred_element_type=jnp.float32)
    eso_ref[...] = jnp.exp(so).astype(BF)
    fso_ref[...] = jnp.exp(NEG * so).astype(BF)
    edot_ref[...] = jnp.exp(do).astype(BF).T
    fdot_ref[...] = jnp.exp(NEG * do).astype(BF).T

    @pl.when(pl.program_id(0) == 0)
    def _():
        csum_ref[...] = jnp.zeros_like(csum_ref)
    csum_ref[...] += jnp.sum(who, axis=0, keepdims=True)


def _attn2_kernel(adj_ref, whoext_ref, eso_ref, fso_ref, edot_ref, fdot_ref,
                  csum_in_ref, out_ref):
    adj = adj_ref[...]
    a = eso_ref[...] * edot_ref[...]
    b = fso_ref[...] * fdot_ref[...]
    p = jnp.maximum(a, b) * adj
    agg_ext = jnp.dot(p, whoext_ref[...], preferred_element_type=jnp.float32)
    agg = agg_ext[:, :HID]
    denom = agg_ext[:, HID:HID + 1]
    whomean = csum_in_ref[...] * (1.0 / N)
    out = jnp.where(denom > 0, agg / denom, whomean)
    out = jnp.where(out > 0, out, jnp.exp(out) - 1.0)
    m2 = jnp.max(out, axis=1, keepdims=True)
    zz = out - m2
    out_ref[...] = zz - jnp.log(jnp.sum(jnp.exp(zz), axis=1, keepdims=True))


def kernel(x, adj, W0, a0, W1, a1, W2, a2, W3, a3, W_out, a_out):
    f32 = jnp.float32
    W_cat = jnp.concatenate([W0, W1, W2, W3], axis=1)
    a_cat = jnp.concatenate([a0, a1, a2, a3], axis=1)

    FH = HEADS * HID
    EXTW = 2 * HID
    whext, es_all, fs_all, edt_all, fdt_all, wh_csum = pl.pallas_call(
        _proj_kernel,
        grid=(N // BLK_A,),
        in_specs=[
            pl.BlockSpec((BLK_A, IN_F), lambda i: (i, 0)),
            pl.BlockSpec((IN_F, FH), lambda i: (0, 0)),
            pl.BlockSpec((2 * HID, HEADS), lambda i: (0, 0)),
        ],
        out_specs=[
            pl.BlockSpec((BLK_A, HEADS * EXTW), lambda i: (i, 0)),
            pl.BlockSpec((BLK_A, HEADS), lambda i: (i, 0)),
            pl.BlockSpec((BLK_A, HEADS), lambda i: (i, 0)),
            pl.BlockSpec((HEADS, BLK_A), lambda i: (0, i)),
            pl.BlockSpec((HEADS, BLK_A), lambda i: (0, i)),
            pl.BlockSpec((1, FH), lambda i: (0, 0)),
        ],
        out_shape=[
            jax.ShapeDtypeStruct((N, HEADS * EXTW), BF),
            jax.ShapeDtypeStruct((N, HEADS), BF),
            jax.ShapeDtypeStruct((N, HEADS), BF),
            jax.ShapeDtypeStruct((HEADS, N), BF),
            jax.ShapeDtypeStruct((HEADS, N), BF),
            jax.ShapeDtypeStruct((1, FH), f32),
        ],
    )(x, W_cat, a_cat)

    adj_bf, whoext, eso, fso, edot, fdot, who_csum = pl.pallas_call(
        _attn1_kernel,
        grid=(N // BLK_B,),
        in_specs=[
            pl.BlockSpec((BLK_B, N), lambda i: (i, 0)),
            pl.BlockSpec((N, HEADS * EXTW), lambda i: (0, 0)),
            pl.BlockSpec((BLK_B, HEADS), lambda i: (i, 0)),
            pl.BlockSpec((BLK_B, HEADS), lambda i: (i, 0)),
            pl.BlockSpec((HEADS, N), lambda i: (0, 0)),
            pl.BlockSpec((HEADS, N), lambda i: (0, 0)),
            pl.BlockSpec((1, FH), lambda i: (0, 0)),
            pl.BlockSpec((FH, HID), lambda i: (0, 0)),
            pl.BlockSpec((2 * HID, 1), lambda i: (0, 0)),
        ],
        out_specs=[
            pl.BlockSpec((BLK_B, N), lambda i: (i, 0)),
            pl.BlockSpec((BLK_B, EXTW), lambda i: (i, 0)),
            pl.BlockSpec((BLK_B, 1), lambda i: (i, 0)),
            pl.BlockSpec((BLK_B, 1), lambda i: (i, 0)),
            pl.BlockSpec((1, BLK_B), lambda i: (0, i)),
            pl.BlockSpec((1, BLK_B), lambda i: (0, i)),
            pl.BlockSpec((1, HID), lambda i: (0, 0)),
        ],
        out_shape=[
            jax.ShapeDtypeStruct((N, N), BF),
            jax.ShapeDtypeStruct((N, EXTW), BF),
            jax.ShapeDtypeStruct((N, 1), BF),
            jax.ShapeDtypeStruct((N, 1), BF),
            jax.ShapeDtypeStruct((1, N), BF),
            jax.ShapeDtypeStruct((1, N), BF),
            jax.ShapeDtypeStruct((1, HID), f32),
        ],
    )(adj, whext, es_all, fs_all, edt_all, fdt_all, wh_csum, W_out, a_out)

    out = pl.pallas_call(
        _attn2_kernel,
        grid=(N // BLK_B,),
        in_specs=[
            pl.BlockSpec((BLK_B, N), lambda i: (i, 0)),
            pl.BlockSpec((N, EXTW), lambda i: (0, 0)),
            pl.BlockSpec((BLK_B, 1), lambda i: (i, 0)),
            pl.BlockSpec((BLK_B, 1), lambda i: (i, 0)),
            pl.BlockSpec((1, N), lambda i: (0, 0)),
            pl.BlockSpec((1, N), lambda i: (0, 0)),
            pl.BlockSpec((1, HID), lambda i: (0, 0)),
        ],
        out_specs=pl.BlockSpec((BLK_B, HID), lambda i: (i, 0)),
        out_shape=jax.ShapeDtypeStruct((N, HID), jnp.float32),
    )(adj_bf, whoext, eso, fso, edot, fdot, who_csum)
    return out


# speedup vs baseline: 3.5342x; 1.0673x over previous
import jax
import jax.numpy as jnp
from jax.experimental import pallas as pl

N = 4096
IN_F = 256
HID = 128
HEADS = 4
NEG = 0.2

BLK_A = 512
BLK_B = 512

BF = jnp.bfloat16


def _proj_kernel(x_ref, w_ref, a_ref,
                 whext_ref, es_ref, fs_ref, edt_ref, fdt_ref, csum_ref):
    x = x_ref[...]
    ext_parts, s_parts, d_parts, wh_parts = [], [], [], []
    blk = x.shape[0]
    for h in range(HEADS):
        wh = jnp.dot(x, w_ref[:, h * HID:(h + 1) * HID],
                     preferred_element_type=jnp.float32)
        wh_parts.append(wh)
        s_parts.append(jnp.dot(wh, a_ref[:HID, h:h + 1],
                               preferred_element_type=jnp.float32))
        d_parts.append(jnp.dot(wh, a_ref[HID:, h:h + 1],
                               preferred_element_type=jnp.float32))
        ext_parts.append(wh.astype(BF))
        ext_parts.append(jnp.ones((blk, 1), BF))
        ext_parts.append(jnp.zeros((blk, HID - 1), BF))
    whext_ref[...] = jnp.concatenate(ext_parts, axis=1)
    s = jnp.concatenate(s_parts, axis=1)
    d = jnp.concatenate(d_parts, axis=1)
    es_ref[...] = jnp.exp(s).astype(BF)
    fs_ref[...] = jnp.exp(NEG * s).astype(BF)
    edt_ref[...] = jnp.exp(d).astype(BF).T
    fdt_ref[...] = jnp.exp(NEG * d).astype(BF).T

    @pl.when(pl.program_id(0) == 0)
    def _():
        csum_ref[...] = jnp.zeros_like(csum_ref)
    csum_ref[...] += jnp.sum(jnp.concatenate(wh_parts, axis=1), axis=0,
                             keepdims=True)


def _attn1_kernel(adj_ref, whext_ref, es_ref, fs_ref, edt_ref, fdt_ref,
                  csum_in_ref, wout_ref, aout_ref,
                  adjb_ref, whoext_ref, eso_ref, fso_ref, edot_ref, fdot_ref,
                  csum_ref):
    adj = adj_ref[...].astype(BF)
    adjb_ref[...] = adj
    whmean = csum_in_ref[...] * (1.0 / N)
    EXTW = 2 * HID
    h_parts = []
    for hd in range(HEADS):
        a = es_ref[:, hd:hd + 1] * edt_ref[hd:hd + 1, :]
        b = fs_ref[:, hd:hd + 1] * fdt_ref[hd:hd + 1, :]
        p = jnp.maximum(a, b) * adj
        agg_ext = jnp.dot(p, whext_ref[:, hd * EXTW:(hd + 1) * EXTW],
                          preferred_element_type=jnp.float32)
        agg = agg_ext[:, :HID]
        denom = agg_ext[:, HID:HID + 1]
        out = jnp.where(denom > 0, agg / denom,
                        whmean[:, hd * HID:(hd + 1) * HID])
        out = jnp.where(out > 0, out, jnp.exp(out) - 1.0)
        h_parts.append(out)
    hblk = jnp.concatenate(h_parts, axis=1)
    who = jnp.dot(hblk, wout_ref[...], preferred_element_type=jnp.float32)
    blk = who.shape[0]
    whoext_ref[...] = jnp.concatenate(
        [who.astype(BF), jnp.ones((blk, 1), BF), jnp.zeros((blk, HID - 1), BF)],
        axis=1)
    so = jnp.dot(who, aout_ref[:HID, :], preferred_element_type=jnp.float32)
    do = jnp.dot(who, aout_ref[HID:, :], preferred_element_type=jnp.float32)
    eso_ref[...] = jnp.exp(so).astype(BF)
    fso_ref[...] = jnp.exp(NEG * so).astype(BF)
    edot_ref[...] = jnp.exp(do).astype(BF).T
    fdot_ref[...] = jnp.exp(NEG * do).astype(BF).T

    @pl.when(pl.program_id(0) == 0)
    def _():
        csum_ref[...] = jnp.zeros_like(csum_ref)
    csum_ref[...] += jnp.sum(who, axis=0, keepdims=True)


def _attn2_kernel(adj_ref, whoext_ref, eso_ref, fso_ref, edot_ref, fdot_ref,
                  csum_in_ref, out_ref):
    adj = adj_ref[...]
    a = eso_ref[...] * edot_ref[...]
    b = fso_ref[...] * fdot_ref[...]
    p = jnp.maximum(a, b) * adj
    agg_ext = jnp.dot(p, whoext_ref[...], preferred_element_type=jnp.float32)
    agg = agg_ext[:, :HID]
    denom = agg_ext[:, HID:HID + 1]
    whomean = csum_in_ref[...] * (1.0 / N)
    out = jnp.where(denom > 0, agg / denom, whomean)
    out = jnp.where(out > 0, out, jnp.exp(out) - 1.0)
    m2 = jnp.max(out, axis=1, keepdims=True)
    zz = out - m2
    out_ref[...] = zz - jnp.log(jnp.sum(jnp.exp(zz), axis=1, keepdims=True))


def kernel(x, adj, W0, a0, W1, a1, W2, a2, W3, a3, W_out, a_out):
    f32 = jnp.float32
    W_cat = jnp.concatenate([W0, W1, W2, W3], axis=1)
    a_cat = jnp.concatenate([a0, a1, a2, a3], axis=1)

    FH = HEADS * HID
    EXTW = 2 * HID
    whext, es_all, fs_all, edt_all, fdt_all, wh_csum = pl.pallas_call(
        _proj_kernel,
        grid=(N // BLK_A,),
        in_specs=[
            pl.BlockSpec((BLK_A, IN_F), lambda i: (i, 0)),
            pl.BlockSpec((IN_F, FH), lambda i: (0, 0)),
            pl.BlockSpec((2 * HID, HEADS), lambda i: (0, 0)),
        ],
        out_specs=[
            pl.BlockSpec((BLK_A, HEADS * EXTW), lambda i: (i, 0)),
            pl.BlockSpec((BLK_A, HEADS), lambda i: (i, 0)),
            pl.BlockSpec((BLK_A, HEADS), lambda i: (i, 0)),
            pl.BlockSpec((HEADS, BLK_A), lambda i: (0, i)),
            pl.BlockSpec((HEADS, BLK_A), lambda i: (0, i)),
            pl.BlockSpec((1, FH), lambda i: (0, 0)),
        ],
        out_shape=[
            jax.ShapeDtypeStruct((N, HEADS * EXTW), BF),
            jax.ShapeDtypeStruct((N, HEADS), BF),
            jax.ShapeDtypeStruct((N, HEADS), BF),
            jax.ShapeDtypeStruct((HEADS, N), BF),
            jax.ShapeDtypeStruct((HEADS, N), BF),
            jax.ShapeDtypeStruct((1, FH), f32),
        ],
    )(x, W_cat, a_cat)

    adj_bf, whoext, eso, fso, edot, fdot, who_csum = pl.pallas_call(
        _attn1_kernel,
        grid=(N // BLK_B,),
        in_specs=[
            pl.BlockSpec((BLK_B, N), lambda i: (i, 0)),
            pl.BlockSpec((N, HEADS * EXTW), lambda i: (0, 0)),
            pl.BlockSpec((BLK_B, HEADS), lambda i: (i, 0)),
            pl.BlockSpec((BLK_B, HEADS), lambda i: (i, 0)),
            pl.BlockSpec((HEADS, N), lambda i: (0, 0)),
            pl.BlockSpec((HEADS, N), lambda i: (0, 0)),
            pl.BlockSpec((1, FH), lambda i: (0, 0)),
            pl.BlockSpec((FH, HID), lambda i: (0, 0)),
            pl.BlockSpec((2 * HID, 1), lambda i: (0, 0)),
        ],
        out_specs=[
            pl.BlockSpec((BLK_B, N), lambda i: (i, 0)),
            pl.BlockSpec((BLK_B, EXTW), lambda i: (i, 0)),
            pl.BlockSpec((BLK_B, 1), lambda i: (i, 0)),
            pl.BlockSpec((BLK_B, 1), lambda i: (i, 0)),
            pl.BlockSpec((1, BLK_B), lambda i: (0, i)),
            pl.BlockSpec((1, BLK_B), lambda i: (0, i)),
            pl.BlockSpec((1, HID), lambda i: (0, 0)),
        ],
        out_shape=[
            jax.ShapeDtypeStruct((N, N), BF),
            jax.ShapeDtypeStruct((N, EXTW), BF),
            jax.ShapeDtypeStruct((N, 1), BF),
            jax.ShapeDtypeStruct((N, 1), BF),
            jax.ShapeDtypeStruct((1, N), BF),
            jax.ShapeDtypeStruct((1, N), BF),
            jax.ShapeDtypeStruct((1, HID), f32),
        ],
    )(adj, whext, es_all, fs_all, edt_all, fdt_all, wh_csum, W_out, a_out)

    out = pl.pallas_call(
        _attn2_kernel,
        grid=(N // BLK_B,),
        in_specs=[
            pl.BlockSpec((BLK_B, N), lambda i: (i, 0)),
            pl.BlockSpec((N, EXTW), lambda i: (0, 0)),
            pl.BlockSpec((BLK_B, 1), lambda i: (i, 0)),
            pl.BlockSpec((BLK_B, 1), lambda i: (i, 0)),
            pl.BlockSpec((1, N), lambda i: (0, 0)),
            pl.BlockSpec((1, N), lambda i: (0, 0)),
            pl.BlockSpec((1, HID), lambda i: (0, 0)),
        ],
        out_specs=pl.BlockSpec((BLK_B, HID), lambda i: (i, 0)),
        out_shape=jax.ShapeDtypeStruct((N, HID), jnp.float32),
    )(adj_bf, whoext, eso, fso, edot, fdot, who_csum)
    return out


# speedup vs baseline: 3.9587x; 1.1201x over previous
import jax
import jax.numpy as jnp
from jax.experimental import pallas as pl
from jax.experimental.pallas import tpu as pltpu

N = 4096
IN_F = 256
HID = 128
HEADS = 4
NEG = 0.2

BLK_B = 512

BF = jnp.bfloat16


def _attn1_kernel(adj_ref, x_ref, w_ref, a_ref, wout_ref, aout_ref,
                  adjb_ref, whoext_ref, eso_ref, fso_ref, edot_ref, fdot_ref,
                  csum_ref,
                  whext_s, es_s, fs_s, edt_s, fdt_s, whmean_s):
    EXTW = 2 * HID

    @pl.when(pl.program_id(0) == 0)
    def _prologue():
        x = x_ref[...]
        ext_parts, s_parts, d_parts, mean_parts = [], [], [], []
        for h in range(HEADS):
            wh = jnp.dot(x, w_ref[:, h * HID:(h + 1) * HID],
                         preferred_element_type=jnp.float32)
            s_parts.append(jnp.dot(wh, a_ref[:HID, h:h + 1],
                                   preferred_element_type=jnp.float32))
            d_parts.append(jnp.dot(wh, a_ref[HID:, h:h + 1],
                                   preferred_element_type=jnp.float32))
            mean_parts.append(jnp.mean(wh, axis=0, keepdims=True))
            ext_parts.append(wh.astype(BF))
            ext_parts.append(jnp.ones((N, 1), BF))
            ext_parts.append(jnp.zeros((N, HID - 1), BF))
        whext_s[...] = jnp.concatenate(ext_parts, axis=1)
        whmean_s[...] = jnp.concatenate(mean_parts, axis=1)
        s = jnp.concatenate(s_parts, axis=1)
        d = jnp.concatenate(d_parts, axis=1)
        es_s[...] = jnp.exp(s).astype(BF)
        fs_s[...] = jnp.exp(NEG * s).astype(BF)
        edt_s[...] = jnp.exp(d).astype(BF).T
        fdt_s[...] = jnp.exp(NEG * d).astype(BF).T

    i = pl.program_id(0)
    adj = adj_ref[...].astype(BF)
    adjb_ref[...] = adj
    row0 = i * BLK_B
    h_parts = []
    for hd in range(HEADS):
        a = es_s[pl.ds(row0, BLK_B), hd:hd + 1] * edt_s[hd:hd + 1, :]
        b = fs_s[pl.ds(row0, BLK_B), hd:hd + 1] * fdt_s[hd:hd + 1, :]
        p = jnp.maximum(a, b) * adj
        agg_ext = jnp.dot(p, whext_s[:, hd * EXTW:(hd + 1) * EXTW],
                          preferred_element_type=jnp.float32)
        agg = agg_ext[:, :HID]
        denom = agg_ext[:, HID:HID + 1]
        out = jnp.where(denom > 0, agg / denom,
                        whmean_s[:, hd * HID:(hd + 1) * HID])
        out = jnp.where(out > 0, out, jnp.exp(out) - 1.0)
        h_parts.append(out)
    hblk = jnp.concatenate(h_parts, axis=1)
    who = jnp.dot(hblk, wout_ref[...], preferred_element_type=jnp.float32)
    whoext_ref[...] = jnp.concatenate(
        [who.astype(BF), jnp.ones((BLK_B, 1), BF),
         jnp.zeros((BLK_B, HID - 1), BF)], axis=1)
    so = jnp.dot(who, aout_ref[:HID, :], preferred_element_type=jnp.float32)
    do = jnp.dot(who, aout_ref[HID:, :], preferred_element_type=jnp.float32)
    eso_ref[...] = jnp.exp(so).astype(BF)
    fso_ref[...] = jnp.exp(NEG * so).astype(BF)
    edot_ref[...] = jnp.exp(do).astype(BF).T
    fdot_ref[...] = jnp.exp(NEG * do).astype(BF).T

    @pl.when(i == 0)
    def _():
        csum_ref[...] = jnp.zeros_like(csum_ref)
    csum_ref[...] += jnp.sum(who, axis=0, keepdims=True)


def _attn2_kernel(adj_ref, whoext_ref, eso_ref, fso_ref, edot_ref, fdot_ref,
                  csum_in_ref, out_ref):
    adj = adj_ref[...]
    a = eso_ref[...] * edot_ref[...]
    b = fso_ref[...] * fdot_ref[...]
    p = jnp.maximum(a, b) * adj
    agg_ext = jnp.dot(p, whoext_ref[...], preferred_element_type=jnp.float32)
    agg = agg_ext[:, :HID]
    denom = agg_ext[:, HID:HID + 1]
    whomean = csum_in_ref[...] * (1.0 / N)
    out = jnp.where(denom > 0, agg / denom, whomean)
    out = jnp.where(out > 0, out, jnp.exp(out) - 1.0)
    m2 = jnp.max(out, axis=1, keepdims=True)
    zz = out - m2
    out_ref[...] = zz - jnp.log(jnp.sum(jnp.exp(zz), axis=1, keepdims=True))


def kernel(x, adj, W0, a0, W1, a1, W2, a2, W3, a3, W_out, a_out):
    f32 = jnp.float32
    W_cat = jnp.concatenate([W0, W1, W2, W3], axis=1)
    a_cat = jnp.concatenate([a0, a1, a2, a3], axis=1)

    FH = HEADS * HID
    EXTW = 2 * HID
    adj_bf, whoext, eso, fso, edot, fdot, who_csum = pl.pallas_call(
        _attn1_kernel,
        grid=(N // BLK_B,),
        in_specs=[
            pl.BlockSpec((BLK_B, N), lambda i: (i, 0)),
            pl.BlockSpec((N, IN_F), lambda i: (0, 0)),
            pl.BlockSpec((IN_F, FH), lambda i: (0, 0)),
            pl.BlockSpec((2 * HID, HEADS), lambda i: (0, 0)),
            pl.BlockSpec((FH, HID), lambda i: (0, 0)),
            pl.BlockSpec((2 * HID, 1), lambda i: (0, 0)),
        ],
        out_specs=[
            pl.BlockSpec((BLK_B, N), lambda i: (i, 0)),
            pl.BlockSpec((BLK_B, EXTW), lambda i: (i, 0)),
            pl.BlockSpec((BLK_B, 1), lambda i: (i, 0)),
            pl.BlockSpec((BLK_B, 1), lambda i: (i, 0)),
            pl.BlockSpec((1, BLK_B), lambda i: (0, i)),
            pl.BlockSpec((1, BLK_B), lambda i: (0, i)),
            pl.BlockSpec((1, HID), lambda i: (0, 0)),
        ],
        out_shape=[
            jax.ShapeDtypeStruct((N, N), BF),
            jax.ShapeDtypeStruct((N, EXTW), BF),
            jax.ShapeDtypeStruct((N, 1), BF),
            jax.ShapeDtypeStruct((N, 1), BF),
            jax.ShapeDtypeStruct((1, N), BF),
            jax.ShapeDtypeStruct((1, N), BF),
            jax.ShapeDtypeStruct((1, HID), f32),
        ],
        scratch_shapes=[
            pltpu.VMEM((N, HEADS * EXTW), BF),
            pltpu.VMEM((N, HEADS), BF),
            pltpu.VMEM((N, HEADS), BF),
            pltpu.VMEM((HEADS, N), BF),
            pltpu.VMEM((HEADS, N), BF),
            pltpu.VMEM((1, FH), f32),
        ],
    )(adj, x, W_cat, a_cat, W_out, a_out)

    out = pl.pallas_call(
        _attn2_kernel,
        grid=(N // BLK_B,),
        in_specs=[
            pl.BlockSpec((BLK_B, N), lambda i: (i, 0)),
            pl.BlockSpec((N, EXTW), lambda i: (0, 0)),
            pl.BlockSpec((BLK_B, 1), lambda i: (i, 0)),
            pl.BlockSpec((BLK_B, 1), lambda i: (i, 0)),
            pl.BlockSpec((1, N), lambda i: (0, 0)),
            pl.BlockSpec((1, N), lambda i: (0, 0)),
            pl.BlockSpec((1, HID), lambda i: (0, 0)),
        ],
        out_specs=pl.BlockSpec((BLK_B, HID), lambda i: (i, 0)),
        out_shape=jax.ShapeDtypeStruct((N, HID), jnp.float32),
    )(adj_bf, whoext, eso, fso, edot, fdot, who_csum)
    return out


# speedup vs baseline: 4.1022x; 1.0363x over previous
import jax
import jax.numpy as jnp
from jax.experimental import pallas as pl
from jax.experimental.pallas import tpu as pltpu

N = 4096
IN_F = 256
HID = 128
HEADS = 4
NEG = 0.2

BLK_B = 512

BF = jnp.bfloat16


def _attn1_kernel(adj_ref, x_ref, w_ref, a_ref, wout_ref, aout_ref,
                  adjb_ref, whoext_ref, eso_ref, fso_ref, edot_ref, fdot_ref,
                  csum_ref,
                  whext_s, es_s, fs_s, edt_s, fdt_s, whmean_s):
    EXTW = 2 * HID

    @pl.when(pl.program_id(0) == 0)
    def _prologue():
        x = x_ref[...]
        ext_parts, s_parts, d_parts, mean_parts = [], [], [], []
        for h in range(HEADS):
            wh = jnp.dot(x, w_ref[:, h * HID:(h + 1) * HID],
                         preferred_element_type=jnp.float32)
            s_parts.append(jnp.dot(wh, a_ref[:HID, h:h + 1],
                                   preferred_element_type=jnp.float32))
            d_parts.append(jnp.dot(wh, a_ref[HID:, h:h + 1],
                                   preferred_element_type=jnp.float32))
            mean_parts.append(jnp.mean(wh, axis=0, keepdims=True))
            ext_parts.append(wh.astype(BF))
            ext_parts.append(jnp.ones((N, 1), BF))
            ext_parts.append(jnp.zeros((N, HID - 1), BF))
        whext_s[...] = jnp.concatenate(ext_parts, axis=1)
        whmean_s[...] = jnp.concatenate(mean_parts, axis=1)
        s = jnp.concatenate(s_parts, axis=1)
        d = jnp.concatenate(d_parts, axis=1)
        es_s[...] = jnp.exp(s).astype(BF)
        fs_s[...] = jnp.exp(NEG * s).astype(BF)
        edt_s[...] = jnp.exp(d).astype(BF).T
        fdt_s[...] = jnp.exp(NEG * d).astype(BF).T

    i = pl.program_id(0)
    adj = adj_ref[...].astype(BF)
    adjb_ref[...] = adj.astype(jnp.int8)
    row0 = i * BLK_B
    h_parts = []
    for hd in range(HEADS):
        a = es_s[pl.ds(row0, BLK_B), hd:hd + 1] * edt_s[hd:hd + 1, :]
        b = fs_s[pl.ds(row0, BLK_B), hd:hd + 1] * fdt_s[hd:hd + 1, :]
        p = jnp.maximum(a, b) * adj
        agg_ext = jnp.dot(p, whext_s[:, hd * EXTW:(hd + 1) * EXTW],
                          preferred_element_type=jnp.float32)
        agg = agg_ext[:, :HID]
        denom = agg_ext[:, HID:HID + 1]
        out = jnp.where(denom > 0, agg / denom,
                        whmean_s[:, hd * HID:(hd + 1) * HID])
        out = jnp.where(out > 0, out, jnp.exp(out) - 1.0)
        h_parts.append(out)
    hblk = jnp.concatenate(h_parts, axis=1)
    who = jnp.dot(hblk, wout_ref[...], preferred_element_type=jnp.float32)
    whoext_ref[...] = jnp.concatenate(
        [who.astype(BF), jnp.ones((BLK_B, 1), BF),
         jnp.zeros((BLK_B, HID - 1), BF)], axis=1)
    so = jnp.dot(who, aout_ref[:HID, :], preferred_element_type=jnp.float32)
    do = jnp.dot(who, aout_ref[HID:, :], preferred_element_type=jnp.float32)
    eso_ref[...] = jnp.exp(so).astype(BF)
    fso_ref[...] = jnp.exp(NEG * so).astype(BF)
    edot_ref[...] = jnp.exp(do).astype(BF).T
    fdot_ref[...] = jnp.exp(NEG * do).astype(BF).T

    @pl.when(i == 0)
    def _():
        csum_ref[...] = jnp.zeros_like(csum_ref)
    csum_ref[...] += jnp.sum(who, axis=0, keepdims=True)


def _attn2_kernel(adj_ref, whoext_ref, eso_ref, fso_ref, edot_ref, fdot_ref,
                  csum_in_ref, out_ref):
    adj = adj_ref[...].astype(BF)
    a = eso_ref[...] * edot_ref[...]
    b = fso_ref[...] * fdot_ref[...]
    p = jnp.maximum(a, b) * adj
    agg_ext = jnp.dot(p, whoext_ref[...], preferred_element_type=jnp.float32)
    agg = agg_ext[:, :HID]
    denom = agg_ext[:, HID:HID + 1]
    whomean = csum_in_ref[...] * (1.0 / N)
    out = jnp.where(denom > 0, agg / denom, whomean)
    out = jnp.where(out > 0, out, jnp.exp(out) - 1.0)
    m2 = jnp.max(out, axis=1, keepdims=True)
    zz = out - m2
    out_ref[...] = zz - jnp.log(jnp.sum(jnp.exp(zz), axis=1, keepdims=True))


def kernel(x, adj, W0, a0, W1, a1, W2, a2, W3, a3, W_out, a_out):
    f32 = jnp.float32
    W_cat = jnp.concatenate([W0, W1, W2, W3], axis=1)
    a_cat = jnp.concatenate([a0, a1, a2, a3], axis=1)

    FH = HEADS * HID
    EXTW = 2 * HID
    adj_bf, whoext, eso, fso, edot, fdot, who_csum = pl.pallas_call(
        _attn1_kernel,
        grid=(N // BLK_B,),
        in_specs=[
            pl.BlockSpec((BLK_B, N), lambda i: (i, 0)),
            pl.BlockSpec((N, IN_F), lambda i: (0, 0)),
            pl.BlockSpec((IN_F, FH), lambda i: (0, 0)),
            pl.BlockSpec((2 * HID, HEADS), lambda i: (0, 0)),
            pl.BlockSpec((FH, HID), lambda i: (0, 0)),
            pl.BlockSpec((2 * HID, 1), lambda i: (0, 0)),
        ],
        out_specs=[
            pl.BlockSpec((BLK_B, N), lambda i: (i, 0)),
            pl.BlockSpec((BLK_B, EXTW), lambda i: (i, 0)),
            pl.BlockSpec((BLK_B, 1), lambda i: (i, 0)),
            pl.BlockSpec((BLK_B, 1), lambda i: (i, 0)),
            pl.BlockSpec((1, BLK_B), lambda i: (0, i)),
            pl.BlockSpec((1, BLK_B), lambda i: (0, i)),
            pl.BlockSpec((1, HID), lambda i: (0, 0)),
        ],
        out_shape=[
            jax.ShapeDtypeStruct((N, N), jnp.int8),
            jax.ShapeDtypeStruct((N, EXTW), BF),
            jax.ShapeDtypeStruct((N, 1), BF),
            jax.ShapeDtypeStruct((N, 1), BF),
            jax.ShapeDtypeStruct((1, N), BF),
            jax.ShapeDtypeStruct((1, N), BF),
            jax.ShapeDtypeStruct((1, HID), f32),
        ],
        scratch_shapes=[
            pltpu.VMEM((N, HEADS * EXTW), BF),
            pltpu.VMEM((N, HEADS), BF),
            pltpu.VMEM((N, HEADS), BF),
            pltpu.VMEM((HEADS, N), BF),
            pltpu.VMEM((HEADS, N), BF),
            pltpu.VMEM((1, FH), f32),
        ],
    )(adj, x, W_cat, a_cat, W_out, a_out)

    out = pl.pallas_call(
        _attn2_kernel,
        grid=(N // BLK_B,),
        in_specs=[
            pl.BlockSpec((BLK_B, N), lambda i: (i, 0)),
            pl.BlockSpec((N, EXTW), lambda i: (0, 0)),
            pl.BlockSpec((BLK_B, 1), lambda i: (i, 0)),
            pl.BlockSpec((BLK_B, 1), lambda i: (i, 0)),
            pl.BlockSpec((1, N), lambda i: (0, 0)),
            pl.BlockSpec((1, N), lambda i: (0, 0)),
            pl.BlockSpec((1, HID), lambda i: (0, 0)),
        ],
        out_specs=pl.BlockSpec((BLK_B, HID), lambda i: (i, 0)),
        out_shape=jax.ShapeDtypeStruct((N, HID), jnp.float32),
    )(adj_bf, whoext, eso, fso, edot, fdot, who_csum)
    return out
